# baseline (speedup 1.0000x reference)
import jax
import jax.numpy as jnp
from jax.experimental import pallas as pl

_C = 50
_H = 256
_E = _C * _C
_TY = 4
_SEM = 57

_R_WOUT = 0
_R_BP = 256
_R_WSEM = 306
_R_WEE = 363
_R_PAR = 367
_R_WEX = 368
_R_BEL = 369
_R_BC = 370
_R_BOUT = 371
_R_BNE = 372
_R_SMALL = 374


def _decoder_kernel(wp_ref, p256_ref, wel_ref, wc_ref, wne_ref,
                    o_cf_ref, o_sem_ref, o_ce_ref, o_eel_ref):
    f32 = jnp.float32

    def dot(a, b):
        return jnp.dot(a, b, preferred_element_type=f32)

    def dott(a, b):
        return jax.lax.dot_general(a, b, (((1,), (1,)), ((), ())),
                                   preferred_element_type=f32)

    def dotT(a, b):
        return jax.lax.dot_general(a, b, (((0,), (0,)), ((), ())),
                                   preferred_element_type=f32)

    parent = p256_ref[_R_PAR:_R_PAR + 1, :]
    wex = p256_ref[_R_WEX:_R_WEX + 1, :]
    bel = p256_ref[_R_BEL:_R_BEL + 1, :]
    bc = p256_ref[_R_BC:_R_BC + 1, :]
    bout = p256_ref[_R_BOUT:_R_BOUT + 1, :]
    bee = p256_ref[_R_SMALL:_R_SMALL + 1, 0:_TY]
    bsem = p256_ref[_R_SMALL:_R_SMALL + 1, _TY:_TY + _SEM]
    bex = p256_ref[_R_SMALL:_R_SMALL + 1, _TY + _SEM:_TY + _SEM + 1]
    bp2 = p256_ref[_R_BP:_R_BP + _C, :]
    wee = p256_ref[_R_WEE:_R_WEE + _TY, :]
    wsem = p256_ref[_R_WSEM:_R_WSEM + _SEM, :]
    wout = p256_ref[_R_WOUT:_R_WOUT + _H, :]

    pf = dott(parent, wp_ref[...])
    cf0 = jnp.maximum(pf.reshape(_C, _H) + bp2, 0.0)

    ce_log = jnp.sum(cf0 * wex, axis=1, keepdims=True) + bex[0, 0]
    o_ce_ref[...] = ce_log
    ce_f = (ce_log > 0.0).astype(f32)

    ei = jax.lax.broadcasted_iota(jnp.int32, (_E, _C), 0)
    ci = jax.lax.broadcasted_iota(jnp.int32, (_E, _C), 1)
    q = ei // _C
    Rm = (q == ci).astype(f32)
    eg = jax.lax.broadcasted_iota(jnp.int32, (_E, 2 * _C), 0)
    cg = jax.lax.broadcasted_iota(jnp.int32, (_E, 2 * _C), 1)
    qg = eg // _C
    G = jnp.where(cg < _C, (qg == cg).astype(f32),
                  (eg - qg * _C == cg - _C).astype(f32))

    wel = wel_ref[...]
    PQ = jnp.concatenate([dott(cf0, wel[:, :_H]),
                          dott(cf0, wel[:, _H:])], axis=0)
    EL = jnp.maximum(dot(G, PQ) + bel, 0.0)

    EEL = dott(EL, wee) + bee
    o_eel_ref[...] = EEL

    ce2 = jnp.concatenate([ce_f, ce_f], axis=0)
    pair = (dot(G, ce2) > 1.5).astype(f32)
    maskf = (EEL > 0.0).astype(f32) * pair
    rowm = jnp.sum(maskf, axis=1, keepdims=True)
    counts = dotT(Rm, rowm)
    has_edges = jnp.sum(counts) > 0.0
    inv = 1.0 / jnp.maximum(counts, 1.0)

    cf = cf0
    cfs = [cf0]
    for it in range(2):
        w = wne_ref[it]
        bne = p256_ref[_R_BNE + it:_R_BNE + it + 1, :]
        AB = jnp.concatenate([dott(cf, w[:, 0:_H]),
                              dott(cf, w[:, _H:2 * _H])], axis=0)
        Cm = dott(EL, w[:, 2 * _H:3 * _H])
        wt = w[:, 3 * _H:]
        base = dot(G, AB) + Cm + bne
        acc = jnp.zeros((_E, _H), dtype=f32)
        for t in range(_TY):
            v = jnp.maximum(base + EEL[:, t:t + 1] * wt[:, t], 0.0)
            acc = acc + maskf[:, t:t + 1] * v
        sums = dotT(Rm, acc)
        cf = jnp.where(has_edges, sums * inv, cf)
        cfs.append(cf)

    wc = wc_ref[...]
    h = jnp.maximum(dott(cfs[0], wc[:, 0:_H]) + dott(cfs[1], wc[:, _H:2 * _H]) +
                    dott(cfs[2], wc[:, 2 * _H:]) + bc, 0.0)
    o_sem_ref[...] = dott(h, wsem) + bsem
    o_cf_ref[...] = jnp.maximum(dott(h, wout) + bout, 0.0)


def kernel(parent_feature, W_parent, b_parent, W_exists, b_exists,
           W_edge_latent, b_edge_latent, W_edge_exists, b_edge_exists,
           W_node_edge, b_node_edge, W_child, b_child, W_sem, b_sem,
           W_child2, b_child2):
    f32 = jnp.float32

    small = jnp.pad(
        jnp.concatenate([b_edge_exists.reshape(1, _TY),
                         b_sem.reshape(1, _SEM),
                         b_exists.reshape(1, 1)], axis=1),
        ((0, 0), (0, _H - _TY - _SEM - 1)))
    p256 = jnp.concatenate([
        W_child2,
        b_parent.reshape(_C, _H),
        W_sem,
        W_edge_exists.reshape(_TY, _H),
        parent_feature,
        W_exists,
        b_edge_latent.reshape(1, _H),
        b_child.reshape(1, _H),
        b_child2.reshape(1, _H),
        b_node_edge,
        small,
    ], axis=0)

    out_shape = (
        jax.ShapeDtypeStruct((_C, _H), f32),
        jax.ShapeDtypeStruct((_C, _SEM), f32),
        jax.ShapeDtypeStruct((_C, 1), f32),
        jax.ShapeDtypeStruct((_E, _TY), f32),
    )
    o_cf, o_sem, o_ce, o_eel = pl.pallas_call(
        _decoder_kernel,
        out_shape=out_shape,
    )(W_parent, p256, W_edge_latent, W_child, W_node_edge)

    return (o_cf.reshape(1, _C, _H),
            o_sem.reshape(1, _C, _SEM),
            o_ce.reshape(1, _C, 1),
            o_eel.reshape(1, _C, _C, _TY))


# speedup vs baseline: 1.1851x; 1.1851x over previous
import jax
import jax.numpy as jnp
from jax.experimental import pallas as pl

_C = 50
_H = 256
_E = _C * _C
_TY = 4
_SEM = 57

def _decoder_kernel(parent_ref, wp_ref, bp_ref, wex_ref, bex_ref,
                    wel_ref, bel_ref, wee_ref, bee_ref,
                    wne_ref, bne_ref,
                    wc_ref, bc_ref, wsem_ref, bsem_ref, wout_ref, bout_ref,
                    o_cf_ref, o_sem_ref, o_ce_ref, o_eel_ref):
    f32 = jnp.float32

    def dot(a, b):
        return jnp.dot(a, b, preferred_element_type=f32)

    def dott(a, b):
        return jax.lax.dot_general(a, b, (((1,), (1,)), ((), ())),
                                   preferred_element_type=f32)

    def dotT(a, b):
        return jax.lax.dot_general(a, b, (((0,), (0,)), ((), ())),
                                   preferred_element_type=f32)

    pf = jnp.maximum(dott(parent_ref[...], wp_ref[...]) + bp_ref[...], 0.0)
    cf0 = pf.reshape(_C, _H)

    ce_log = jnp.sum(cf0 * wex_ref[...], axis=1, keepdims=True) + bex_ref[0, 0]
    o_ce_ref[...] = ce_log
    ce_f = (ce_log > 0.0).astype(f32)

    ei = jax.lax.broadcasted_iota(jnp.int32, (_E, _C), 0)
    ci = jax.lax.broadcasted_iota(jnp.int32, (_E, _C), 1)
    q = ei // _C
    Rm = (q == ci).astype(f32)
    eg = jax.lax.broadcasted_iota(jnp.int32, (_E, 2 * _C), 0)
    cg = jax.lax.broadcasted_iota(jnp.int32, (_E, 2 * _C), 1)
    qg = eg // _C
    G = jnp.where(cg < _C, (qg == cg).astype(f32),
                  (eg - qg * _C == cg - _C).astype(f32))

    wel = wel_ref[...]
    PQ = jnp.concatenate([dott(cf0, wel[:, :_H]),
                          dott(cf0, wel[:, _H:])], axis=0)
    EL = jnp.maximum(dot(G, PQ) + bel_ref[...], 0.0)

    EEL = dott(EL, wee_ref[...]) + bee_ref[...]
    o_eel_ref[...] = EEL

    ce2 = jnp.concatenate([ce_f, ce_f], axis=0)
    pair = (dot(G, ce2) > 1.5).astype(f32)
    maskf = (EEL > 0.0).astype(f32) * pair
    rowm = jnp.sum(maskf, axis=1, keepdims=True)
    counts = dotT(Rm, rowm)
    has_edges = jnp.sum(counts) > 0.0
    inv = 1.0 / jnp.maximum(counts, 1.0)

    cf = cf0
    cfs = [cf0]
    for it in range(2):
        w = wne_ref[it]
        AB = jnp.concatenate([dott(cf, w[:, 0:_H]),
                              dott(cf, w[:, _H:2 * _H])], axis=0)
        Cm = dott(EL, w[:, 2 * _H:3 * _H])
        wt = w[:, 3 * _H:]
        base = dot(G, AB) + Cm + bne_ref[it:it + 1, :]
        acc = jnp.zeros((_E, _H), dtype=f32)
        for t in range(_TY):
            v = jnp.maximum(base + EEL[:, t:t + 1] * wt[:, t], 0.0)
            acc = acc + maskf[:, t:t + 1] * v
        sums = dotT(Rm, acc)
        cf = jnp.where(has_edges, sums * inv, cf)
        cfs.append(cf)

    wc = wc_ref[...]
    h = jnp.maximum(dott(cfs[0], wc[:, 0:_H]) + dott(cfs[1], wc[:, _H:2 * _H]) +
                    dott(cfs[2], wc[:, 2 * _H:]) + bc_ref[...], 0.0)
    o_sem_ref[...] = dott(h, wsem_ref[...]) + bsem_ref[...]
    o_cf_ref[...] = jnp.maximum(dott(h, wout_ref[...]) + bout_ref[...], 0.0)


def kernel(parent_feature, W_parent, b_parent, W_exists, b_exists,
           W_edge_latent, b_edge_latent, W_edge_exists, b_edge_exists,
           W_node_edge, b_node_edge, W_child, b_child, W_sem, b_sem,
           W_child2, b_child2):
    f32 = jnp.float32
    args = [
        parent_feature,
        W_parent,
        b_parent.reshape(1, -1),
        W_exists,
        b_exists.reshape(1, 1),
        W_edge_latent,
        b_edge_latent.reshape(1, _H),
        W_edge_exists.reshape(_TY, _H),
        b_edge_exists.reshape(1, _TY),
        W_node_edge,
        b_node_edge,
        W_child,
        b_child.reshape(1, _H),
        W_sem,
        b_sem.reshape(1, _SEM),
        W_child2,
        b_child2.reshape(1, _H),
    ]

    out_shape = (
        jax.ShapeDtypeStruct((_C, _H), f32),
        jax.ShapeDtypeStruct((_C, _SEM), f32),
        jax.ShapeDtypeStruct((_C, 1), f32),
        jax.ShapeDtypeStruct((_E, _TY), f32),
    )
    o_cf, o_sem, o_ce, o_eel = pl.pallas_call(
        _decoder_kernel,
        out_shape=out_shape,
    )(*args)

    return (o_cf.reshape(1, _C, _H),
            o_sem.reshape(1, _C, _SEM),
            o_ce.reshape(1, _C, 1),
            o_eel.reshape(1, _C, _C, _TY))
